# baseline (speedup 1.0000x reference)
import functools

import jax
import jax.numpy as jnp
from jax import lax
from jax.experimental import pallas as pl
from jax.experimental.pallas import tpu as pltpu
from jax.experimental.pallas import tpu_sc as plsc

_VOCAB = 1000000
_L = 200
_B = 4096
_D = 64
_TOPICS = 50
_PAD_T = 128

_NC = 2
_NS = 16
_NW = _NC * _NS
_SPW = _B // _NW
_LANES = 16

_VB = 128
_NFULL = _VOCAB // _VB
_TAIL = _VOCAB - _NFULL * _VB
_NIT = 246
_PITCH = 72

_CH0 = 128
_CH1 = _L - _CH0


def _pool_body(idx_hbm, table_hbm, out_hbm, idx_v, buf0, buf1, sums_v, sem0, sem1):
    wid = lax.axis_index("s") * _NC + lax.axis_index("c")
    base = wid * _SPW
    pltpu.sync_copy(idx_hbm.at[pl.ds(base * _L, _SPW * _L)], idx_v)

    bufs = (buf0, buf1)
    sems = (sem0, sem1)

    def start_gather(s, which):
        buf = bufs[which]
        sem = sems[which]
        off = pl.multiple_of(s * _L, 8)
        c0 = pltpu.make_async_copy(
            table_hbm.at[idx_v.at[pl.ds(off, _CH0)]], buf.at[pl.ds(0, _CH0)], sem)
        c1 = pltpu.make_async_copy(
            table_hbm.at[idx_v.at[pl.ds(off + _CH0, _CH1)]], buf.at[pl.ds(_CH0, _CH1)], sem)
        c0.start()
        c1.start()

    def wait_gather(which):
        buf = bufs[which]
        sem = sems[which]
        pltpu.make_async_copy(
            table_hbm.at[idx_v.at[pl.ds(0, _CH0)]], buf.at[pl.ds(0, _CH0)], sem).wait()
        pltpu.make_async_copy(
            table_hbm.at[idx_v.at[pl.ds(0, _CH1)]], buf.at[pl.ds(_CH0, _CH1)], sem).wait()

    def accumulate(s, which):
        buf = bufs[which]

        def rbody(r, accs):
            return tuple(a + buf[r, pl.ds(c * _LANES, _LANES)]
                         for c, a in enumerate(accs))

        accs = lax.fori_loop(
            0, _L, rbody,
            tuple(jnp.zeros((_LANES,), jnp.float32) for _ in range(_D // _LANES)))
        for c, a in enumerate(accs):
            sums_v[s, pl.ds(c * _LANES, _LANES)] = a * (1.0 / _L)

    start_gather(0, 0)

    def pair_body(p, _):
        s0 = p * 2
        start_gather(s0 + 1, 1)
        wait_gather(0)
        accumulate(s0, 0)

        @pl.when(s0 + 2 < _SPW)
        def _():
            start_gather(s0 + 2, 0)

        wait_gather(1)
        accumulate(s0 + 1, 1)
        return 0

    lax.fori_loop(0, _SPW // 2, pair_body, 0)
    pltpu.sync_copy(sums_v, out_hbm.at[pl.ds(base, _SPW)])


@functools.cache
def _get_pool():
    return pl.kernel(
        _pool_body,
        out_type=jax.ShapeDtypeStruct((_B, _D), jnp.float32),
        mesh=plsc.VectorSubcoreMesh(core_axis_name="c", subcore_axis_name="s"),
        scratch_types=[
            pltpu.VMEM((_SPW * _L,), jnp.int32),
            pltpu.VMEM((_L, _D), jnp.float32),
            pltpu.VMEM((_L, _D), jnp.float32),
            pltpu.VMEM((_SPW, _D), jnp.float32),
            pltpu.SemaphoreType.DMA,
            pltpu.SemaphoreType.DMA,
        ],
        compiler_params=pltpu.CompilerParams(use_tc_tiling_on_sc=False),
    )


_TCOLS = 512
_TGRID = -(-_VOCAB // _TCOLS)


def _tr_tc_body(tt_ref, out_ref):
    t = jnp.transpose(tt_ref[...])
    out_ref[...] = jnp.concatenate([t, t], axis=1)


def _tr_tc(tt):
    return pl.pallas_call(
        _tr_tc_body,
        grid=(_TGRID,),
        in_specs=[pl.BlockSpec((_D, _TCOLS), lambda i: (0, i))],
        out_specs=pl.BlockSpec((_TCOLS, 2 * _D), lambda i: (i, 0)),
        out_shape=jax.ShapeDtypeStruct((_TGRID * _TCOLS, 2 * _D), jnp.float32),
    )(tt)


def _mlp_body(x_ref, w1_ref, b1_ref, w2_ref, b2_ref, wc_ref, bc_ref, out_ref):
    x = x_ref[...]
    h = jnp.maximum(jnp.dot(x, w1_ref[...],
                            preferred_element_type=jnp.float32) + b1_ref[...], 0.0)
    h = jnp.maximum(jnp.dot(h, w2_ref[...],
                            preferred_element_type=jnp.float32) + b2_ref[...], 0.0)
    z = jnp.dot(h, wc_ref[...], preferred_element_type=jnp.float32) + bc_ref[...]
    out_ref[...] = 1.0 / (1.0 + jnp.exp(-z))


def _mlp(x, w1, b1, w2, b2, wc, bc):
    return pl.pallas_call(
        _mlp_body,
        out_shape=jax.ShapeDtypeStruct((_B, _PAD_T), jnp.float32),
    )(x, w1, b1, w2, b2, wc, bc)


def kernel(inputs, table, W1, b1, W2, b2, Wc, bc):
    idx2 = inputs.reshape(-1).astype(jnp.int32) * 2
    lin = _tr_tc(table.T).reshape(2 * _TGRID * _TCOLS, _D)
    pooled = _get_pool()(idx2, lin)
    wc_p = jnp.pad(Wc, ((0, 0), (0, _PAD_T - _TOPICS)))
    bc_p = jnp.pad(bc, (0, _PAD_T - _TOPICS))
    out = _mlp(pooled, W1, b1.reshape(1, -1), W2, b2.reshape(1, -1),
               wc_p, bc_p.reshape(1, -1))
    return out[:, :_TOPICS]


# speedup vs baseline: 2.2042x; 2.2042x over previous
import functools

import jax
import jax.numpy as jnp
from jax import lax
from jax.experimental import pallas as pl
from jax.experimental.pallas import tpu as pltpu
from jax.experimental.pallas import tpu_sc as plsc

_VOCAB = 1000000
_L = 200
_B = 4096
_D = 64
_TOPICS = 50
_PAD_T = 128

_NC = 2
_NS = 16
_NW = _NC * _NS
_SPW = _B // _NW
_LANES = 16

_VB = 128
_NFULL = _VOCAB // _VB
_TAIL = _VOCAB - _NFULL * _VB
_NIT = 246
_PITCH = 72

_CH0 = 128
_CH1 = _L - _CH0


def _pool_body(idx_hbm, table_hbm, out_hbm, idx_v, buf0, buf1, sums_v, sem0, sem1):
    wid = lax.axis_index("s") * _NC + lax.axis_index("c")
    base = wid * _SPW
    pltpu.sync_copy(idx_hbm.at[pl.ds(base * _L, _SPW * _L)], idx_v)

    bufs = (buf0, buf1)
    sems = (sem0, sem1)

    def start_gather(s, which):
        buf = bufs[which]
        sem = sems[which]
        off = pl.multiple_of(s * _L, 8)
        c0 = pltpu.make_async_copy(
            table_hbm.at[idx_v.at[pl.ds(off, _CH0)]], buf.at[pl.ds(0, _CH0)], sem)
        c1 = pltpu.make_async_copy(
            table_hbm.at[idx_v.at[pl.ds(off + _CH0, _CH1)]], buf.at[pl.ds(_CH0, _CH1)], sem)
        c0.start()
        c1.start()

    def wait_gather(which):
        buf = bufs[which]
        sem = sems[which]
        pltpu.make_async_copy(
            table_hbm.at[idx_v.at[pl.ds(0, _CH0)]], buf.at[pl.ds(0, _CH0)], sem).wait()
        pltpu.make_async_copy(
            table_hbm.at[idx_v.at[pl.ds(0, _CH1)]], buf.at[pl.ds(_CH0, _CH1)], sem).wait()

    def accumulate(s, which):
        buf = bufs[which]

        def rbody(r, accs):
            return tuple(a + buf[r, pl.ds(c * _LANES, _LANES)]
                         for c, a in enumerate(accs))

        accs = lax.fori_loop(
            0, _L, rbody,
            tuple(jnp.zeros((_LANES,), jnp.float32) for _ in range(_D // _LANES)))
        for c, a in enumerate(accs):
            sums_v[s, pl.ds(c * _LANES, _LANES)] = a * (1.0 / _L)

    start_gather(0, 0)

    def pair_body(p, _):
        s0 = p * 2
        start_gather(s0 + 1, 1)
        wait_gather(0)
        accumulate(s0, 0)

        @pl.when(s0 + 2 < _SPW)
        def _():
            start_gather(s0 + 2, 0)

        wait_gather(1)
        accumulate(s0 + 1, 1)
        return 0

    lax.fori_loop(0, _SPW // 2, pair_body, 0)
    pltpu.sync_copy(sums_v, out_hbm.at[pl.ds(base, _SPW)])


@functools.cache
def _get_pool():
    return pl.kernel(
        _pool_body,
        out_type=jax.ShapeDtypeStruct((_B, _D), jnp.float32),
        mesh=plsc.VectorSubcoreMesh(core_axis_name="c", subcore_axis_name="s"),
        scratch_types=[
            pltpu.VMEM((_SPW * _L,), jnp.int32),
            pltpu.VMEM((_L, _D), jnp.float32),
            pltpu.VMEM((_L, _D), jnp.float32),
            pltpu.VMEM((_SPW, _D), jnp.float32),
            pltpu.SemaphoreType.DMA,
            pltpu.SemaphoreType.DMA,
        ],
        compiler_params=pltpu.CompilerParams(use_tc_tiling_on_sc=False),
    )


_TCOLS = 2048
_TGRID = -(-_VOCAB // _TCOLS)


def _tr_tc_body(tt_ref, i2_ref, out_ref):
    out_ref[...] = lax.dot_general(
        tt_ref[...], i2_ref[...], (((0,), (0,)), ((), ())),
        preferred_element_type=jnp.float32)


def _tr_tc(tt, i2):
    return pl.pallas_call(
        _tr_tc_body,
        grid=(_TGRID,),
        in_specs=[pl.BlockSpec((_D, _TCOLS), lambda i: (0, i)),
                  pl.BlockSpec((_D, 2 * _D), lambda i: (0, 0))],
        out_specs=pl.BlockSpec((_TCOLS, 2 * _D), lambda i: (i, 0)),
        out_shape=jax.ShapeDtypeStruct((_TGRID * _TCOLS, 2 * _D), jnp.float32),
    )(tt, i2)


def _mlp_body(x_ref, w1_ref, b1_ref, w2_ref, b2_ref, wc_ref, bc_ref, out_ref):
    x = x_ref[...]
    h = jnp.maximum(jnp.dot(x, w1_ref[...],
                            preferred_element_type=jnp.float32) + b1_ref[...], 0.0)
    h = jnp.maximum(jnp.dot(h, w2_ref[...],
                            preferred_element_type=jnp.float32) + b2_ref[...], 0.0)
    z = jnp.dot(h, wc_ref[...], preferred_element_type=jnp.float32) + bc_ref[...]
    out_ref[...] = 1.0 / (1.0 + jnp.exp(-z))


def _mlp(x, w1, b1, w2, b2, wc, bc):
    return pl.pallas_call(
        _mlp_body,
        out_shape=jax.ShapeDtypeStruct((_B, _PAD_T), jnp.float32),
    )(x, w1, b1, w2, b2, wc, bc)


def kernel(inputs, table, W1, b1, W2, b2, Wc, bc):
    idx2 = inputs.reshape(-1).astype(jnp.int32) * 2
    eye = jnp.eye(_D, dtype=jnp.float32)
    i2 = jnp.concatenate([eye, eye], axis=1)
    lin = _tr_tc(table.T, i2).reshape(2 * _TGRID * _TCOLS, _D)
    pooled = _get_pool()(idx2, lin)
    wc_p = jnp.pad(Wc, ((0, 0), (0, _PAD_T - _TOPICS)))
    bc_p = jnp.pad(bc, (0, _PAD_T - _TOPICS))
    out = _mlp(pooled, W1, b1.reshape(1, -1), W2, b2.reshape(1, -1),
               wc_p, bc_p.reshape(1, -1))
    return out[:, :_TOPICS]


# speedup vs baseline: 2.9458x; 1.3364x over previous
import functools

import jax
import jax.numpy as jnp
from jax import lax
from jax.experimental import pallas as pl
from jax.experimental.pallas import tpu as pltpu
from jax.experimental.pallas import tpu_sc as plsc

_VOCAB = 1000000
_L = 200
_B = 4096
_D = 64
_TOPICS = 50
_PAD_T = 128

_NC = 2
_NS = 16
_NW = _NC * _NS
_SPW = _B // _NW
_LANES = 16

_VB = 128
_NFULL = _VOCAB // _VB
_TAIL = _VOCAB - _NFULL * _VB
_NIT = 246
_PITCH = 72

_CH0 = 128
_CH1 = _L - _CH0


def _pool_body(idx_hbm, table_hbm, out_hbm, idx_v, buf0, buf1, sums_v, sem0, sem1):
    wid = lax.axis_index("s") * _NC + lax.axis_index("c")
    base = wid * _SPW
    pltpu.sync_copy(idx_hbm.at[pl.ds(base * _L, _SPW * _L)], idx_v)

    bufs = (buf0, buf1)
    sems = (sem0, sem1)

    def start_gather(s, which):
        buf = bufs[which]
        sem = sems[which]
        off = pl.multiple_of(s * _L, 8)
        c0 = pltpu.make_async_copy(
            table_hbm.at[idx_v.at[pl.ds(off, _CH0)]], buf.at[pl.ds(0, _CH0)], sem)
        c1 = pltpu.make_async_copy(
            table_hbm.at[idx_v.at[pl.ds(off + _CH0, _CH1)]], buf.at[pl.ds(_CH0, _CH1)], sem)
        c0.start()
        c1.start()

    def wait_gather(which):
        buf = bufs[which]
        sem = sems[which]
        pltpu.make_async_copy(
            table_hbm.at[idx_v.at[pl.ds(0, _CH0)]], buf.at[pl.ds(0, _CH0)], sem).wait()
        pltpu.make_async_copy(
            table_hbm.at[idx_v.at[pl.ds(0, _CH1)]], buf.at[pl.ds(_CH0, _CH1)], sem).wait()

    def accumulate(s, which):
        buf = bufs[which]

        def rbody(r, accs):
            return tuple(a + buf[r, pl.ds(c * _LANES, _LANES)]
                         for c, a in enumerate(accs))

        accs = lax.fori_loop(
            0, _L, rbody,
            tuple(jnp.zeros((_LANES,), jnp.float32) for _ in range(_D // _LANES)))
        for c, a in enumerate(accs):
            sums_v[s, pl.ds(c * _LANES, _LANES)] = a * (1.0 / _L)

    start_gather(0, 0)

    def pair_body(p, _):
        s0 = p * 2
        start_gather(s0 + 1, 1)
        wait_gather(0)
        accumulate(s0, 0)

        @pl.when(s0 + 2 < _SPW)
        def _():
            start_gather(s0 + 2, 0)

        wait_gather(1)
        accumulate(s0 + 1, 1)
        return 0

    lax.fori_loop(0, _SPW // 2, pair_body, 0)
    pltpu.sync_copy(sums_v, out_hbm.at[pl.ds(base, _SPW)])


@functools.cache
def _get_pool():
    return pl.kernel(
        _pool_body,
        out_type=jax.ShapeDtypeStruct((_B, _D), jnp.float32),
        mesh=plsc.VectorSubcoreMesh(core_axis_name="c", subcore_axis_name="s"),
        scratch_types=[
            pltpu.VMEM((_SPW * _L,), jnp.int32),
            pltpu.VMEM((_L, _D), jnp.float32),
            pltpu.VMEM((_L, _D), jnp.float32),
            pltpu.VMEM((_SPW, _D), jnp.float32),
            pltpu.SemaphoreType.DMA,
            pltpu.SemaphoreType.DMA,
        ],
        compiler_params=pltpu.CompilerParams(use_tc_tiling_on_sc=False),
    )


_TCOLS = 8192
_TGRID = -(-_VOCAB // _TCOLS)


def _tr_tc_body(tt_ref, i2_ref, out_ref):
    t = lax.dot_general(
        tt_ref[...], i2_ref[...], (((0,), (0,)), ((), ())),
        preferred_element_type=jnp.float32)
    out_ref[...] = jnp.concatenate([t, t], axis=1)


def _tr_tc(tt, i2):
    return pl.pallas_call(
        _tr_tc_body,
        grid=(_TGRID,),
        in_specs=[pl.BlockSpec((_D, _TCOLS), lambda i: (0, i)),
                  pl.BlockSpec((_D, _D), lambda i: (0, 0))],
        out_specs=pl.BlockSpec((_TCOLS, 2 * _D), lambda i: (i, 0)),
        out_shape=jax.ShapeDtypeStruct((_TGRID * _TCOLS, 2 * _D), jnp.float32),
    )(tt, i2)


def _mlp_body(x_ref, w1_ref, b1_ref, w2_ref, b2_ref, wc_ref, bc_ref, out_ref):
    x = x_ref[...]
    h = jnp.maximum(jnp.dot(x, w1_ref[...],
                            preferred_element_type=jnp.float32) + b1_ref[...], 0.0)
    h = jnp.maximum(jnp.dot(h, w2_ref[...],
                            preferred_element_type=jnp.float32) + b2_ref[...], 0.0)
    z = jnp.dot(h, wc_ref[...], preferred_element_type=jnp.float32) + bc_ref[...]
    out_ref[...] = 1.0 / (1.0 + jnp.exp(-z))


def _mlp(x, w1, b1, w2, b2, wc, bc):
    return pl.pallas_call(
        _mlp_body,
        out_shape=jax.ShapeDtypeStruct((_B, _PAD_T), jnp.float32),
    )(x, w1, b1, w2, b2, wc, bc)


def kernel(inputs, table, W1, b1, W2, b2, Wc, bc):
    idx2 = inputs.reshape(-1).astype(jnp.int32) * 2
    eye = jnp.eye(_D, dtype=jnp.float32)
    lin = _tr_tc(table.T, eye).reshape(2 * _TGRID * _TCOLS, _D)
    pooled = _get_pool()(idx2, lin)
    wc_p = jnp.pad(Wc, ((0, 0), (0, _PAD_T - _TOPICS)))
    bc_p = jnp.pad(bc, (0, _PAD_T - _TOPICS))
    out = _mlp(pooled, W1, b1.reshape(1, -1), W2, b2.reshape(1, -1),
               wc_p, bc_p.reshape(1, -1))
    return out[:, :_TOPICS]


# speedup vs baseline: 3.1621x; 1.0734x over previous
import functools

import jax
import jax.numpy as jnp
from jax import lax
from jax.experimental import pallas as pl
from jax.experimental.pallas import tpu as pltpu
from jax.experimental.pallas import tpu_sc as plsc

_VOCAB = 1000000
_L = 200
_B = 4096
_D = 64
_TOPICS = 50
_PAD_T = 128

_NC = 2
_NS = 16
_NW = _NC * _NS
_SPW = _B // _NW
_LANES = 16

_VB = 128
_NFULL = _VOCAB // _VB
_TAIL = _VOCAB - _NFULL * _VB
_NIT = 246
_PITCH = 72

_CH0 = 128
_CH1 = _L - _CH0


def _pool_body(idx_hbm, table_hbm, out_hbm, idx_v, buf0, buf1, sums_v, sem0, sem1):
    wid = lax.axis_index("s") * _NC + lax.axis_index("c")
    base = wid * _SPW
    pltpu.sync_copy(idx_hbm.at[pl.ds(base * _L, _SPW * _L)], idx_v)

    bufs = (buf0, buf1)
    sems = (sem0, sem1)

    def start_gather(s, which):
        buf = bufs[which]
        sem = sems[which]
        off = pl.multiple_of(s * _L, 8)
        c0 = pltpu.make_async_copy(
            table_hbm.at[idx_v.at[pl.ds(off, _CH0)]], buf.at[pl.ds(0, _CH0)], sem)
        c1 = pltpu.make_async_copy(
            table_hbm.at[idx_v.at[pl.ds(off + _CH0, _CH1)]], buf.at[pl.ds(_CH0, _CH1)], sem)
        c0.start()
        c1.start()

    def wait_gather(which):
        buf = bufs[which]
        sem = sems[which]
        pltpu.make_async_copy(
            table_hbm.at[idx_v.at[pl.ds(0, _CH0)]], buf.at[pl.ds(0, _CH0)], sem).wait()
        pltpu.make_async_copy(
            table_hbm.at[idx_v.at[pl.ds(0, _CH1)]], buf.at[pl.ds(_CH0, _CH1)], sem).wait()

    def accumulate(s, which):
        buf = bufs[which]

        def rbody(r, accs):
            return tuple(a + buf[r, pl.ds(c * _LANES, _LANES)]
                         for c, a in enumerate(accs))

        accs = lax.fori_loop(
            0, _L, rbody,
            tuple(jnp.zeros((_LANES,), jnp.float32) for _ in range(_D // _LANES)))
        for c, a in enumerate(accs):
            sums_v[s, pl.ds(c * _LANES, _LANES)] = a * (1.0 / _L)

    start_gather(0, 0)

    def pair_body(p, _):
        s0 = p * 2
        start_gather(s0 + 1, 1)
        wait_gather(0)
        accumulate(s0, 0)

        @pl.when(s0 + 2 < _SPW)
        def _():
            start_gather(s0 + 2, 0)

        wait_gather(1)
        accumulate(s0 + 1, 1)
        return 0

    lax.fori_loop(0, _SPW // 2, pair_body, 0)
    pltpu.sync_copy(sums_v, out_hbm.at[pl.ds(base, _SPW)])


@functools.cache
def _get_pool():
    return pl.kernel(
        _pool_body,
        out_type=jax.ShapeDtypeStruct((_B, _D), jnp.float32),
        mesh=plsc.VectorSubcoreMesh(core_axis_name="c", subcore_axis_name="s"),
        scratch_types=[
            pltpu.VMEM((_SPW * _L,), jnp.int32),
            pltpu.VMEM((_L, _D), jnp.float32),
            pltpu.VMEM((_L, _D), jnp.float32),
            pltpu.VMEM((_SPW, _D), jnp.float32),
            pltpu.SemaphoreType.DMA,
            pltpu.SemaphoreType.DMA,
        ],
        compiler_params=pltpu.CompilerParams(use_tc_tiling_on_sc=False),
    )


_TCOLS = 16384
_TGRID = -(-_VOCAB // _TCOLS)


def _tr_tc_body(tt_ref, i2_ref, out_ref):
    t = lax.dot_general(
        tt_ref[...], i2_ref[...], (((0,), (0,)), ((), ())),
        preferred_element_type=jnp.float32)
    out_ref[...] = jnp.concatenate([t, t], axis=1)


def _tr_tc(tt, i2):
    return pl.pallas_call(
        _tr_tc_body,
        grid=(_TGRID,),
        in_specs=[pl.BlockSpec((_D, _TCOLS), lambda i: (0, i)),
                  pl.BlockSpec((_D, _D), lambda i: (0, 0))],
        out_specs=pl.BlockSpec((_TCOLS, 2 * _D), lambda i: (i, 0)),
        out_shape=jax.ShapeDtypeStruct((_TGRID * _TCOLS, 2 * _D), jnp.float32),
    )(tt, i2)


def _mlp_body(x_ref, w1_ref, b1_ref, w2_ref, b2_ref, wc_ref, bc_ref, out_ref):
    x = x_ref[...]
    h = jnp.maximum(jnp.dot(x, w1_ref[...],
                            preferred_element_type=jnp.float32) + b1_ref[...], 0.0)
    h = jnp.maximum(jnp.dot(h, w2_ref[...],
                            preferred_element_type=jnp.float32) + b2_ref[...], 0.0)
    z = jnp.dot(h, wc_ref[...], preferred_element_type=jnp.float32) + bc_ref[...]
    out_ref[...] = 1.0 / (1.0 + jnp.exp(-z))


def _mlp(x, w1, b1, w2, b2, wc, bc):
    return pl.pallas_call(
        _mlp_body,
        out_shape=jax.ShapeDtypeStruct((_B, _PAD_T), jnp.float32),
    )(x, w1, b1, w2, b2, wc, bc)


def kernel(inputs, table, W1, b1, W2, b2, Wc, bc):
    idx2 = inputs.reshape(-1).astype(jnp.int32) * 2
    eye = jnp.eye(_D, dtype=jnp.float32)
    lin = _tr_tc(table.T, eye).reshape(2 * _TGRID * _TCOLS, _D)
    pooled = _get_pool()(idx2, lin)
    wc_p = jnp.pad(Wc, ((0, 0), (0, _PAD_T - _TOPICS)))
    bc_p = jnp.pad(bc, (0, _PAD_T - _TOPICS))
    out = _mlp(pooled, W1, b1.reshape(1, -1), W2, b2.reshape(1, -1),
               wc_p, bc_p.reshape(1, -1))
    return out[:, :_TOPICS]
